# baseline (speedup 1.0000x reference)
import functools

import jax
import jax.numpy as jnp
from jax import lax
from jax.experimental import pallas as pl
from jax.experimental.pallas import tpu as pltpu
from jax.experimental.pallas import tpu_sc as plsc

F32 = jnp.float32


def _pack_halves(xf32):
    Dh = xf32.shape[1] // 2
    lo = xf32[:, :Dh].astype(jnp.bfloat16)
    hi = xf32[:, Dh:].astype(jnp.bfloat16)
    lo_i = lax.bitcast_convert_type(lo, jnp.uint16).astype(jnp.int32)
    hi_i = lax.bitcast_convert_type(hi, jnp.uint16).astype(jnp.int32)
    return lax.bitcast_convert_type(lo_i | (hi_i << 16), F32)


def _pq_call(x, WdT, WsT):
    N, D = x.shape
    BN = 2000

    def body(x_ref, wd_ref, ws_ref, p_ref, q_ref):
        xb = x_ref[...]
        p_ref[...] = jnp.dot(xb, wd_ref[...], preferred_element_type=F32)
        q_ref[...] = jnp.dot(xb, ws_ref[...], preferred_element_type=F32)

    return pl.pallas_call(
        body,
        grid=(N // BN,),
        in_specs=[pl.BlockSpec((BN, D), lambda i: (i, 0)),
                  pl.BlockSpec((D, D), lambda i: (0, 0)),
                  pl.BlockSpec((D, D), lambda i: (0, 0))],
        out_specs=[pl.BlockSpec((BN, D), lambda i: (i, 0)),
                   pl.BlockSpec((BN, D), lambda i: (i, 0))],
        out_shape=[jax.ShapeDtypeStruct((N, D), F32)] * 2,
    )(x, WdT, WsT)


def _eh_call(ea, WeT, b1):
    E, DE = ea.shape
    D = WeT.shape[1]
    BE = 8000

    def body(ea_ref, wr, br, out_r):
        eb = ea_ref[...]
        out_r[...] = jnp.dot(eb, wr[...], preferred_element_type=F32) + br[...]

    return pl.pallas_call(
        body,
        grid=(E // BE,),
        in_specs=[pl.BlockSpec((BE, DE), lambda i: (i, 0)),
                  pl.BlockSpec((DE, D), lambda i: (0, 0)),
                  pl.BlockSpec((1, D), lambda i: (0, 0))],
        out_specs=pl.BlockSpec((BE, D), lambda i: (i, 0)),
        out_shape=jax.ShapeDtypeStruct((E, D), F32),
    )(ea, WeT, b1)


def _sc_edge_pass(P, Q, Eh, dst2d, src2d, zeroA, zeroD, nch, with_deg):
    N, D = P.shape
    CPAD, CB = dst2d.shape
    NCH = nch
    assert CB == 32 and D % 16 == 0 and N % 16 == 0
    RPT = N // 16
    KPT = -(-NCH // 32)
    assert CPAD >= (31 * (NCH // 32) + min(31, NCH % 32)) + KPT

    IH = 84
    RELOAD_K = 78
    assert NCH // 32 >= IH + 2 and KPT - IH <= IH and RELOAD_K + 4 < IH

    mesh = plsc.VectorSubcoreMesh(core_axis_name="c", subcore_axis_name="s")
    out_types = [jax.ShapeDtypeStruct((2, N, D), F32)]
    if with_deg:
        out_types.append(jax.ShapeDtypeStruct((2, N, 16), F32))
    scratch = [
        pltpu.VMEM((2, CB, D), F32),
        pltpu.VMEM((2, CB, D), F32),
        pltpu.VMEM((2, CB, D), F32),
        pltpu.VMEM((2, CB, D), F32),
        pltpu.VMEM((IH, CB), jnp.int32),
        pltpu.VMEM((IH, CB), jnp.int32),
        pltpu.VMEM((CB, 16), F32),
        pltpu.SemaphoreType.DMA,
        pltpu.SemaphoreType.DMA,
        pltpu.SemaphoreType.DMA,
        pltpu.SemaphoreType.DMA,
        pltpu.SemaphoreType.DMA,
        pltpu.SemaphoreType.DMA,
        pltpu.SemaphoreType.DMA,
        pltpu.SemaphoreType.DMA,
        pltpu.SemaphoreType.DMA,
        pltpu.SemaphoreType.DMA,
        pltpu.SemaphoreType.DMA,
        pltpu.SemaphoreType.DMA,
        pltpu.VMEM_SHARED((N, D), F32),
        pltpu.VMEM_SHARED((N, 16), F32),
    ]

    def body(P_hbm, Q_hbm, Eh_hbm, dst_hbm, src_hbm, zA_hbm, zD_hbm,
             A_out, *rest):
        if with_deg:
            (Deg_out, Pd, Qs, Ehb, Tb, dst_all, src_all, ones16,
             sp0, sp1, sq0, sq1, se0, se1, sa0, sa1, sd0, sd1, si0, si1,
             A_sh, Deg_sh) = rest
        else:
            (Pd, Qs, Ehb, Tb, dst_all, src_all, ones16,
             sp0, sp1, sq0, sq1, se0, se1, sa0, sa1, sd0, sd1, si0, si1,
             A_sh, Deg_sh) = rest
            Deg_out = None
        sems = ((sp0, sq0, se0), (sp1, sq1, se1))
        ssca = (sa0, sa1)
        sscd = (sd0, sd1)
        cid = lax.axis_index("c")
        sid = lax.axis_index("s")
        wid = sid * 2 + cid

        nbase = NCH // 32
        extra = NCH % 32
        cnt = nbase + jnp.where(wid < extra, 1, 0)
        cstart = wid * nbase + jnp.minimum(wid, extra)

        pltpu.sync_copy(dst_hbm.at[pl.ds(cstart, IH)], dst_all)
        pltpu.sync_copy(src_hbm.at[pl.ds(cstart, IH)], src_all)

        def fill_row(r, carry):
            iota = lax.iota(jnp.int32, 16)
            ones16[r, pl.ds(0, 16)] = jnp.where(iota == 0, 1.0, 0.0).astype(F32)
            return carry
        if with_deg:
            lax.fori_loop(0, CB, fill_row, 0)

        base_row = sid * RPT
        pltpu.sync_copy(zA_hbm.at[pl.ds(base_row, RPT)],
                        A_sh.at[pl.ds(base_row, RPT)])
        if with_deg:
            pltpu.sync_copy(zD_hbm.at[pl.ds(base_row, RPT)],
                            Deg_sh.at[pl.ds(base_row, RPT)])
        plsc.subcore_barrier()

        IR = KPT - IH

        def idxrow(k):
            return jnp.where(k < IH, k, k - IH)

        def issue(k, b):
            sp, sq, se = sems[b]
            r = idxrow(k)
            pltpu.async_copy(P_hbm.at[dst_all.at[r]], Pd.at[b], sp)
            pltpu.async_copy(Q_hbm.at[src_all.at[r]], Qs.at[b], sq)
            pltpu.async_copy(Eh_hbm.at[pl.ds((cstart + k) * CB, CB)],
                             Ehb.at[b], se)

        def wait(k, b):
            sp, sq, se = sems[b]
            r = idxrow(k)
            pltpu.make_async_copy(P_hbm.at[dst_all.at[r]], Pd.at[b], sp).wait()
            pltpu.make_async_copy(Q_hbm.at[src_all.at[r]], Qs.at[b], sq).wait()
            pltpu.make_async_copy(Eh_hbm.at[pl.ds((cstart + k) * CB, CB)],
                                  Ehb.at[b], se).wait()

        issue(0, 0)

        def pair_body(i, carry):
            for b in range(2):
                k = 2 * i + b

                @pl.when(k < cnt)
                def _():
                    wait(k, b)

                    @pl.when(k + 1 == IH)
                    def _():
                        pltpu.make_async_copy(
                            dst_hbm.at[pl.ds(cstart + IH, IR)],
                            dst_all.at[pl.ds(0, IR)], si0).wait()
                        pltpu.make_async_copy(
                            src_hbm.at[pl.ds(cstart + IH, IR)],
                            src_all.at[pl.ds(0, IR)], si1).wait()

                    @pl.when(k + 1 < cnt)
                    def _():
                        issue(k + 1, 1 - b)

                    @pl.when(k >= 2)
                    def _():
                        pltpu.make_async_copy(
                            Tb.at[b], A_sh.at[dst_all.at[0]], ssca[b]).wait()
                        if with_deg:
                            pltpu.make_async_copy(
                                ones16, Deg_sh.at[dst_all.at[0]],
                                sscd[b]).wait()

                    @plsc.parallel_loop(0, CB, unroll=4)
                    def rbody(r):
                        for g in range(D // 16):
                            sl = pl.ds(g * 16, 16)
                            Tb[b, r, sl] = jnp.maximum(
                                Pd[b, r, sl] + Qs[b, r, sl] + Ehb[b, r, sl],
                                0.0)

                    ir = idxrow(k)
                    pltpu.async_copy(Tb.at[b], A_sh.at[dst_all.at[ir]],
                                     ssca[b])
                    if with_deg:
                        pltpu.async_copy(ones16, Deg_sh.at[dst_all.at[ir]],
                                         sscd[b])

                    @pl.when(k == RELOAD_K)
                    def _():
                        pltpu.async_copy(dst_hbm.at[pl.ds(cstart + IH, IR)],
                                         dst_all.at[pl.ds(0, IR)], si0)
                        pltpu.async_copy(src_hbm.at[pl.ds(cstart + IH, IR)],
                                         src_all.at[pl.ds(0, IR)], si1)
            return carry
        lax.fori_loop(0, (cnt + 1) // 2, pair_body, 0)
        for b in range(2):
            pltpu.make_async_copy(
                Tb.at[b], A_sh.at[dst_all.at[0]], ssca[b]).wait()
            if with_deg:
                pltpu.make_async_copy(
                    ones16, Deg_sh.at[dst_all.at[0]], sscd[b]).wait()
        plsc.subcore_barrier()

        pltpu.sync_copy(A_sh.at[pl.ds(base_row, RPT)],
                        A_out.at[cid, pl.ds(base_row, RPT)])
        if with_deg:
            pltpu.sync_copy(Deg_sh.at[pl.ds(base_row, RPT)],
                            Deg_out.at[cid, pl.ds(base_row, RPT)])

    run = pl.kernel(body, mesh=mesh, out_type=out_types, scratch_types=scratch,
                    compiler_params=pltpu.CompilerParams(
                        use_tc_tiling_on_sc=False,
                        needs_layout_passes=False))
    return run(P, Q, Eh, dst2d, src2d, zeroA, zeroD)


def _gr_update(x, A_part, Deg_part, W2T, b2, WihT, WhhT, bsum,
               WgT, bgp, WfT, bfp, WdT2=None, WsT2=None):
    N, D = x.shape
    BN = 2000
    GP = WgT.shape[1]
    emit_pq = WdT2 is not None
    nsteps = N // BN

    def body(x_ref, a2_ref, dg_ref, w2t, b2r, wih, whh, bs, wgt, bg, wft, bf,
             *refs):
        if emit_pq:
            wd2, ws2, x1_ref, hg_ref, p_ref, q_ref, acc = refs
        else:
            x1_ref, hg_ref, acc = refs
        i = pl.program_id(0)
        A = a2_ref[0] + a2_ref[1]
        deg = dg_ref[0, :, 0:1] + dg_ref[1, :, 0:1]
        a = jnp.dot(A, w2t[...], preferred_element_type=F32) + deg * b2r[...]
        xb = x_ref[...]
        g = (jnp.dot(xb, wih[...], preferred_element_type=F32)
             + jnp.dot(a, whh[...], preferred_element_type=F32) + bs[...])
        i_g = g[:, 0:D]
        g_g = g[:, 2 * D:3 * D]
        o_g = g[:, 3 * D:4 * D]
        c = jax.nn.sigmoid(i_g) * jnp.tanh(g_g)
        xn = jax.nn.sigmoid(o_g) * jnp.tanh(c)
        x1_ref[...] = xn
        if emit_pq:
            p_ref[...] = jnp.dot(xn, wd2[...], preferred_element_type=F32)
            q_ref[...] = jnp.dot(xn, ws2[...], preferred_element_type=F32)
        gate = jax.nn.sigmoid(jnp.dot(xn, wgt[...], preferred_element_type=F32) + bg[...])
        hv = jnp.dot(xn, wft[...], preferred_element_type=F32) + bf[...]
        contrib = jnp.sum(gate * hv, axis=0, keepdims=True)

        @pl.when(i == 0)
        def _():
            acc[...] = jnp.zeros_like(acc)

        acc[0:1, 0:GP] += contrib
        hg_ref[...] = acc[0:1, 0:GP]

    in_specs = [
        pl.BlockSpec((BN, D), lambda i: (i, 0)),
        pl.BlockSpec((2, BN, D), lambda i: (0, i, 0)),
        pl.BlockSpec((2, BN, 16), lambda i: (0, i, 0)),
        pl.BlockSpec((D, D), lambda i: (0, 0)),
        pl.BlockSpec((1, D), lambda i: (0, 0)),
        pl.BlockSpec((D, 4 * D), lambda i: (0, 0)),
        pl.BlockSpec((D, 4 * D), lambda i: (0, 0)),
        pl.BlockSpec((1, 4 * D), lambda i: (0, 0)),
        pl.BlockSpec((D, GP), lambda i: (0, 0)),
        pl.BlockSpec((1, GP), lambda i: (0, 0)),
        pl.BlockSpec((D, GP), lambda i: (0, 0)),
        pl.BlockSpec((1, GP), lambda i: (0, 0)),
    ]
    args = [x, A_part, Deg_part, W2T, b2, WihT, WhhT, bsum, WgT, bgp, WfT, bfp]
    out_specs = [pl.BlockSpec((BN, D), lambda i: (i, 0)),
                 pl.BlockSpec((1, GP), lambda i: (0, 0))]
    out_shape = [jax.ShapeDtypeStruct((N, D), F32),
                 jax.ShapeDtypeStruct((1, GP), F32)]
    if emit_pq:
        in_specs += [pl.BlockSpec((D, D), lambda i: (0, 0)),
                     pl.BlockSpec((D, D), lambda i: (0, 0))]
        args += [WdT2, WsT2]
        out_specs += [pl.BlockSpec((BN, D), lambda i: (i, 0)),
                      pl.BlockSpec((BN, D), lambda i: (i, 0))]
        out_shape += [jax.ShapeDtypeStruct((N, D), F32),
                      jax.ShapeDtypeStruct((N, D), F32)]

    return pl.pallas_call(
        body,
        grid=(nsteps,),
        in_specs=in_specs,
        out_specs=out_specs,
        out_shape=out_shape,
        scratch_shapes=[pltpu.VMEM((8, 128), F32)],
    )(*args)


def _head_call(h_G, hG_nn, x2, seq_node, seq_edge,
               fanWT, fan_b, A1, A2, nn_b, B1, B2, fae_b, C1, C2, fs_b):
    N, D = x2.shape

    def _pick(v, idx):
        it = lax.broadcasted_iota(jnp.int32, v.shape, 1)
        return jnp.sum(jnp.where(it == idx, v, 0.0))

    def body(seqn, seqe, hg_ref, hgnn_ref, fanwt, fanb, a1, a2, nnb,
             b1m, b2m, faeb, c1m, c2m, fsb, x2_hbm, out_ref, x2u, sem):
        u = seqe[0]
        ntype = seqn[1]
        bond = seqe[2]
        cp = pltpu.make_async_copy(x2_hbm.at[pl.ds(u, 1)], x2u, sem)
        cp.start()
        hg = hg_ref[...]
        hgnn = hgnn_ref[...]
        ntl = jnp.dot(hg, fanwt[...], preferred_element_type=F32) + fanb[...]
        m1 = jnp.max(ntl)
        p_nt = jnp.exp(ntl - m1)
        p_nt = p_nt / jnp.sum(p_nt)
        lp = jnp.log(_pick(p_nt, ntype)) + jnp.log(_pick(p_nt, 113))
        nne = (jnp.dot(hgnn, a1[...], preferred_element_type=F32)
               + jnp.dot(p_nt, a2[...], preferred_element_type=F32) + nnb[...])
        ael = (jnp.dot(hg, b1m[...], preferred_element_type=F32)
               + jnp.dot(nne, b2m[...], preferred_element_type=F32) + faeb[...])
        m2 = jnp.max(ael)
        p_ae = jnp.exp(ael - m2)
        p_ae = p_ae / jnp.sum(p_ae)
        lp = lp + jnp.log(_pick(p_ae, 1)) + jnp.log(_pick(p_ae, 0))
        cp.wait()
        sl = (jnp.dot(x2u[...], c1m[...], preferred_element_type=F32)
              + jnp.dot(nne, c2m[...], preferred_element_type=F32) + fsb[...])
        m3 = jnp.max(sl)
        p_s = jnp.exp(sl - m3)
        p_s = p_s / jnp.sum(p_s)
        lp = lp + jnp.log(_pick(p_s, bond))
        out_ref[0, 0] = lp

    return pl.pallas_call(
        body,
        in_specs=[
            pl.BlockSpec(memory_space=pltpu.SMEM),
            pl.BlockSpec(memory_space=pltpu.SMEM),
            pl.BlockSpec(memory_space=pltpu.VMEM),
            pl.BlockSpec(memory_space=pltpu.VMEM),
            pl.BlockSpec(memory_space=pltpu.VMEM),
            pl.BlockSpec(memory_space=pltpu.VMEM),
            pl.BlockSpec(memory_space=pltpu.VMEM),
            pl.BlockSpec(memory_space=pltpu.VMEM),
            pl.BlockSpec(memory_space=pltpu.VMEM),
            pl.BlockSpec(memory_space=pltpu.VMEM),
            pl.BlockSpec(memory_space=pltpu.VMEM),
            pl.BlockSpec(memory_space=pltpu.VMEM),
            pl.BlockSpec(memory_space=pltpu.VMEM),
            pl.BlockSpec(memory_space=pltpu.VMEM),
            pl.BlockSpec(memory_space=pltpu.VMEM),
            pl.BlockSpec(memory_space=pl.ANY),
        ],
        out_specs=pl.BlockSpec(memory_space=pltpu.SMEM),
        out_shape=jax.ShapeDtypeStruct((1, 1), F32),
        scratch_shapes=[pltpu.VMEM((1, D), F32), pltpu.SemaphoreType.DMA],
    )(seq_node, seq_edge, h_G, hG_nn, fanWT, fan_b, A1, A2, nn_b,
      B1, B2, fae_b, C1, C2, fs_b, x2)


def kernel(x, edge_attr, edge_index, seq_node, seq_edge,
           R_fe_W1, R_fe_b1, R_fe_W2, R_fe_b2, R_Wih, R_Whh, R_bih, R_bhh,
           R_Wg, R_bg, R_Wf, R_bf,
           NN_fe_W1, NN_fe_b1, NN_fe_W2, NN_fe_b2, NN_Wih, NN_Whh, NN_bih,
           NN_bhh, NN_Wg, NN_bg, NN_Wf, NN_bf,
           NE_fe_W1, NE_fe_b1, NE_fe_W2, NE_fe_b2, NE_Wih, NE_Whh, NE_bih,
           NE_bhh, NE_Wg, NE_bg, NE_Wf, NE_bf,
           fan_W, fan_b, fae_W, fae_b, fs_W, fs_b,
           nn_init_W, nn_init_b, ne_init_W, ne_init_b):
    N, D = x.shape
    E, DE = edge_attr.shape
    G = fan_W.shape[1]
    GP = 64

    def prep(W1, b1, W2, Wih, Whh, bih, bhh, Wg, bg, Wf, bf):
        WdT = W1[:, :D].T
        WsT = W1[:, D:2 * D].T
        WeT = W1[:, 2 * D:].T
        b1r = b1[None, :]
        W2T = W2.T
        WihT = Wih.T
        WhhT = Whh.T
        bsum = (bih + bhh)[None, :]
        WgT = jnp.zeros((D, GP), F32).at[:, :G].set(Wg.T)
        bgp = jnp.zeros((1, GP), F32).at[0, :G].set(bg)
        WfT = jnp.zeros((D, GP), F32).at[:, :G].set(Wf.T)
        bfp = jnp.zeros((1, GP), F32).at[0, :G].set(bf)
        return WdT, WsT, WeT, b1r, W2T, WihT, WhhT, bsum, WgT, bgp, WfT, bfp

    (R_WdT, R_WsT, R_WeT, R_b1r, R_W2T, R_WihT, R_WhhT, R_bsum,
     R_WgT, R_bgp, R_WfT, R_bfp) = prep(R_fe_W1, R_fe_b1, R_fe_W2, R_Wih,
                                        R_Whh, R_bih, R_bhh, R_Wg, R_bg,
                                        R_Wf, R_bf)
    (N_WdT, N_WsT, N_WeT, N_b1r, N_W2T, N_WihT, N_WhhT, N_bsum,
     N_WgT, N_bgp, N_WfT, N_bfp) = prep(NN_fe_W1, NN_fe_b1, NN_fe_W2, NN_Wih,
                                        NN_Whh, NN_bih, NN_bhh, NN_Wg, NN_bg,
                                        NN_Wf, NN_bf)

    CB = 32
    NCH = E // CB
    KPT = -(-NCH // 32)
    CPAD = 31 * (NCH // 32) + min(31, NCH % 32) + KPT
    CPAD = -(-CPAD // 8) * 8
    pad = CPAD * CB - E
    dst2d = jnp.concatenate(
        [edge_index[1], jnp.zeros((pad,), jnp.int32)]).reshape(CPAD, CB)
    src2d = jnp.concatenate(
        [edge_index[0], jnp.zeros((pad,), jnp.int32)]).reshape(CPAD, CB)
    zeroA = jnp.zeros((N, D), F32)
    zeroD = jnp.zeros((N, 16), F32)

    P_R, Q_R = _pq_call(x, R_WdT, R_WsT)
    Eh_R = _eh_call(edge_attr, R_WeT, R_b1r)
    A_R, Deg = _sc_edge_pass(P_R, Q_R, Eh_R, dst2d, src2d, zeroA, zeroD,
                             NCH, with_deg=True)
    Eh_NN = _eh_call(edge_attr, N_WeT, N_b1r)
    x1, h_G, P_NN, Q_NN = _gr_update(
        x, A_R, Deg, R_W2T, R_fe_b2[None, :], R_WihT, R_WhhT, R_bsum,
        R_WgT, R_bgp, R_WfT, R_bfp, N_WdT, N_WsT)

    (A_NN,) = _sc_edge_pass(P_NN, Q_NN, Eh_NN, dst2d, src2d, zeroA, zeroD,
                            NCH, with_deg=False)
    x2, hG_nn = _gr_update(
        x1, A_NN, Deg, N_W2T, NN_fe_b2[None, :], N_WihT, N_WhhT, N_bsum,
        N_WgT, N_bgp, N_WfT, N_bfp)

    fanWT = jnp.zeros((GP, 114), F32).at[:G].set(fan_W.T)
    A1 = jnp.zeros((GP, D), F32).at[:G].set(nn_init_W[:, :G].T)
    A2 = nn_init_W[:, G:].T
    B1 = jnp.zeros((GP, 2), F32).at[:G].set(fae_W[:, :G].T)
    B2 = fae_W[:, G:].T
    C1 = fs_W[:, :D].T
    C2 = fs_W[:, D:].T
    lp = _head_call(h_G, hG_nn, x2, seq_node, seq_edge,
                    fanWT, fan_b[None, :], A1, A2, nn_init_b[None, :],
                    B1, B2, fae_b[None, :], C1, C2, fs_b[None, :])
    return lp[0, 0]


# speedup vs baseline: 1.1124x; 1.1124x over previous
import functools

import jax
import jax.numpy as jnp
from jax import lax
from jax.experimental import pallas as pl
from jax.experimental.pallas import tpu as pltpu
from jax.experimental.pallas import tpu_sc as plsc

F32 = jnp.float32


def _pack_halves(xf32):
    Dh = xf32.shape[1] // 2
    lo = xf32[:, :Dh].astype(jnp.bfloat16)
    hi = xf32[:, Dh:].astype(jnp.bfloat16)
    lo_i = lax.bitcast_convert_type(lo, jnp.uint16).astype(jnp.int32)
    hi_i = lax.bitcast_convert_type(hi, jnp.uint16).astype(jnp.int32)
    return lax.bitcast_convert_type(lo_i | (hi_i << 16), F32)


def _pq_call(x, WdT, WsT):
    N, D = x.shape
    BN = 2000

    def body(x_ref, wd_ref, ws_ref, p_ref, q_ref):
        xb = x_ref[...]
        p_ref[...] = jnp.dot(xb, wd_ref[...], preferred_element_type=F32)
        q_ref[...] = jnp.dot(xb, ws_ref[...], preferred_element_type=F32)

    return pl.pallas_call(
        body,
        grid=(N // BN,),
        in_specs=[pl.BlockSpec((BN, D), lambda i: (i, 0)),
                  pl.BlockSpec((D, D), lambda i: (0, 0)),
                  pl.BlockSpec((D, D), lambda i: (0, 0))],
        out_specs=[pl.BlockSpec((BN, D), lambda i: (i, 0)),
                   pl.BlockSpec((BN, D), lambda i: (i, 0))],
        out_shape=[jax.ShapeDtypeStruct((N, D), F32)] * 2,
    )(x, WdT, WsT)


def _eh_call(ea, WeT, b1, packed=False):
    E, DE = ea.shape
    D = WeT.shape[1]
    BE = 8000
    DO = D // 2 if packed else D

    def body(ea_ref, wr, br, out_r):
        eb = ea_ref[...]
        r = jnp.dot(eb, wr[...], preferred_element_type=F32) + br[...]
        out_r[...] = _pack_halves(r) if packed else r

    return pl.pallas_call(
        body,
        grid=(E // BE,),
        in_specs=[pl.BlockSpec((BE, DE), lambda i: (i, 0)),
                  pl.BlockSpec((DE, D), lambda i: (0, 0)),
                  pl.BlockSpec((1, D), lambda i: (0, 0))],
        out_specs=pl.BlockSpec((BE, DO), lambda i: (i, 0)),
        out_shape=jax.ShapeDtypeStruct((E, DO), F32),
    )(ea, WeT, b1)


def _sc_edge_pass(P, Q, Eh, dst2d, src2d, zeroA, zeroD, nch, with_deg):
    N, D = P.shape
    CPAD, CB = dst2d.shape
    NCH = nch
    assert CB == 32 and D % 16 == 0 and N % 16 == 0
    RPT = N // 16
    KPT = -(-NCH // 32)
    assert CPAD >= (31 * (NCH // 32) + min(31, NCH % 32)) + KPT

    IH = 84
    RELOAD_K = 78
    assert NCH // 32 >= IH + 2 and KPT - IH <= IH and RELOAD_K + 4 < IH

    mesh = plsc.VectorSubcoreMesh(core_axis_name="c", subcore_axis_name="s")
    out_types = [jax.ShapeDtypeStruct((2, N, D), F32)]
    if with_deg:
        out_types.append(jax.ShapeDtypeStruct((2, N, 16), F32))
    scratch = [
        pltpu.VMEM((2, CB, D), F32),
        pltpu.VMEM((2, CB, D), F32),
        pltpu.VMEM((2, CB, D), F32),
        pltpu.VMEM((2, CB, D), F32),
        pltpu.VMEM((IH, CB), jnp.int32),
        pltpu.VMEM((IH, CB), jnp.int32),
        pltpu.VMEM((CB, 16), F32),
        pltpu.SemaphoreType.DMA,
        pltpu.SemaphoreType.DMA,
        pltpu.SemaphoreType.DMA,
        pltpu.SemaphoreType.DMA,
        pltpu.SemaphoreType.DMA,
        pltpu.SemaphoreType.DMA,
        pltpu.SemaphoreType.DMA,
        pltpu.SemaphoreType.DMA,
        pltpu.SemaphoreType.DMA,
        pltpu.SemaphoreType.DMA,
        pltpu.SemaphoreType.DMA,
        pltpu.SemaphoreType.DMA,
        pltpu.VMEM_SHARED((N, D), F32),
        pltpu.VMEM_SHARED((N, 16), F32),
    ]

    def body(P_hbm, Q_hbm, Eh_hbm, dst_hbm, src_hbm, zA_hbm, zD_hbm,
             A_out, *rest):
        if with_deg:
            (Deg_out, Pd, Qs, Ehb, Tb, dst_all, src_all, ones16,
             sp0, sp1, sq0, sq1, se0, se1, sa0, sa1, sd0, sd1, si0, si1,
             A_sh, Deg_sh) = rest
        else:
            (Pd, Qs, Ehb, Tb, dst_all, src_all, ones16,
             sp0, sp1, sq0, sq1, se0, se1, sa0, sa1, sd0, sd1, si0, si1,
             A_sh, Deg_sh) = rest
            Deg_out = None
        sems = ((sp0, sq0, se0), (sp1, sq1, se1))
        ssca = (sa0, sa1)
        sscd = (sd0, sd1)
        cid = lax.axis_index("c")
        sid = lax.axis_index("s")
        wid = sid * 2 + cid

        nbase = NCH // 32
        extra = NCH % 32
        cnt = nbase + jnp.where(wid < extra, 1, 0)
        cstart = wid * nbase + jnp.minimum(wid, extra)

        pltpu.sync_copy(dst_hbm.at[pl.ds(cstart, IH)], dst_all)
        pltpu.sync_copy(src_hbm.at[pl.ds(cstart, IH)], src_all)

        def fill_row(r, carry):
            iota = lax.iota(jnp.int32, 16)
            ones16[r, pl.ds(0, 16)] = jnp.where(iota == 0, 1.0, 0.0).astype(F32)
            return carry
        if with_deg:
            lax.fori_loop(0, CB, fill_row, 0)

        base_row = sid * RPT
        pltpu.sync_copy(zA_hbm.at[pl.ds(base_row, RPT)],
                        A_sh.at[pl.ds(base_row, RPT)])
        if with_deg:
            pltpu.sync_copy(zD_hbm.at[pl.ds(base_row, RPT)],
                            Deg_sh.at[pl.ds(base_row, RPT)])
        plsc.subcore_barrier()

        IR = KPT - IH

        def idxrow(k):
            return jnp.where(k < IH, k, k - IH)

        def issue(k, b):
            sp, sq, se = sems[b]
            r = idxrow(k)
            pltpu.async_copy(P_hbm.at[dst_all.at[r]], Pd.at[b], sp)
            pltpu.async_copy(Q_hbm.at[src_all.at[r]], Qs.at[b], sq)
            pltpu.async_copy(Eh_hbm.at[pl.ds((cstart + k) * CB, CB)],
                             Ehb.at[b], se)

        def wait(k, b):
            sp, sq, se = sems[b]
            r = idxrow(k)
            pltpu.make_async_copy(P_hbm.at[dst_all.at[r]], Pd.at[b], sp).wait()
            pltpu.make_async_copy(Q_hbm.at[src_all.at[r]], Qs.at[b], sq).wait()
            pltpu.make_async_copy(Eh_hbm.at[pl.ds((cstart + k) * CB, CB)],
                                  Ehb.at[b], se).wait()

        issue(0, 0)

        def pair_body(i, carry):
            for b in range(2):
                k = 2 * i + b

                @pl.when(k < cnt)
                def _():
                    wait(k, b)

                    @pl.when(k + 1 == IH)
                    def _():
                        pltpu.make_async_copy(
                            dst_hbm.at[pl.ds(cstart + IH, IR)],
                            dst_all.at[pl.ds(0, IR)], si0).wait()
                        pltpu.make_async_copy(
                            src_hbm.at[pl.ds(cstart + IH, IR)],
                            src_all.at[pl.ds(0, IR)], si1).wait()

                    @pl.when(k + 1 < cnt)
                    def _():
                        issue(k + 1, 1 - b)

                    @pl.when(k >= 2)
                    def _():
                        pltpu.make_async_copy(
                            Tb.at[b], A_sh.at[dst_all.at[0]], ssca[b]).wait()
                        if with_deg:
                            pltpu.make_async_copy(
                                ones16, Deg_sh.at[dst_all.at[0]],
                                sscd[b]).wait()

                    @plsc.parallel_loop(0, CB, unroll=4)
                    def rbody(r):
                        for g in range(D // 16):
                            sl = pl.ds(g * 16, 16)
                            Tb[b, r, sl] = jnp.maximum(
                                Pd[b, r, sl] + Qs[b, r, sl] + Ehb[b, r, sl],
                                0.0)

                    ir = idxrow(k)
                    pltpu.async_copy(Tb.at[b], A_sh.at[dst_all.at[ir]],
                                     ssca[b])
                    if with_deg:
                        pltpu.async_copy(ones16, Deg_sh.at[dst_all.at[ir]],
                                         sscd[b])

                    @pl.when(k == RELOAD_K)
                    def _():
                        pltpu.async_copy(dst_hbm.at[pl.ds(cstart + IH, IR)],
                                         dst_all.at[pl.ds(0, IR)], si0)
                        pltpu.async_copy(src_hbm.at[pl.ds(cstart + IH, IR)],
                                         src_all.at[pl.ds(0, IR)], si1)
            return carry
        lax.fori_loop(0, (cnt + 1) // 2, pair_body, 0)
        for b in range(2):
            pltpu.make_async_copy(
                Tb.at[b], A_sh.at[dst_all.at[0]], ssca[b]).wait()
            if with_deg:
                pltpu.make_async_copy(
                    ones16, Deg_sh.at[dst_all.at[0]], sscd[b]).wait()
        plsc.subcore_barrier()

        pltpu.sync_copy(A_sh.at[pl.ds(base_row, RPT)],
                        A_out.at[cid, pl.ds(base_row, RPT)])
        if with_deg:
            pltpu.sync_copy(Deg_sh.at[pl.ds(base_row, RPT)],
                            Deg_out.at[cid, pl.ds(base_row, RPT)])

    run = pl.kernel(body, mesh=mesh, out_type=out_types, scratch_types=scratch,
                    compiler_params=pltpu.CompilerParams(
                        use_tc_tiling_on_sc=False,
                        needs_layout_passes=False))
    return run(P, Q, Eh, dst2d, src2d, zeroA, zeroD)


def _sc_edge_pass_packed(Ppk, Qpk, Ehpk, dst2d, src2d, zeroA, nch):
    N, Dh = Ppk.shape
    D = 2 * Dh
    CPAD, CB = dst2d.shape
    NCH = nch
    assert CB == 64 and D % 16 == 0 and N % 16 == 0
    RPT = N // 16
    KPT = -(-NCH // 32)
    assert CPAD >= (31 * (NCH // 32) + min(31, NCH % 32)) + KPT
    IH = 42
    RELOAD_K = 36
    assert NCH // 32 >= IH + 2 and KPT - IH <= IH and RELOAD_K + 4 < IH

    mesh = plsc.VectorSubcoreMesh(core_axis_name="c", subcore_axis_name="s")
    scratch = [
        pltpu.VMEM((2, CB, Dh), F32),
        pltpu.VMEM((2, CB, Dh), F32),
        pltpu.VMEM((2, CB, Dh), F32),
        pltpu.VMEM((2, CB, D), F32),
        pltpu.VMEM((IH, CB), jnp.int32),
        pltpu.VMEM((IH, CB), jnp.int32),
        pltpu.SemaphoreType.DMA,
        pltpu.SemaphoreType.DMA,
        pltpu.SemaphoreType.DMA,
        pltpu.SemaphoreType.DMA,
        pltpu.SemaphoreType.DMA,
        pltpu.SemaphoreType.DMA,
        pltpu.SemaphoreType.DMA,
        pltpu.SemaphoreType.DMA,
        pltpu.SemaphoreType.DMA,
        pltpu.SemaphoreType.DMA,
        pltpu.VMEM_SHARED((N, D), F32),
    ]

    def body(P_hbm, Q_hbm, Eh_hbm, dst_hbm, src_hbm, zA_hbm, A_out,
             Pd, Qs, Ehb, Tb, dst_all, src_all,
             sp0, sp1, sq0, sq1, se0, se1, sa0, sa1, si0, si1, A_sh):
        sems = ((sp0, sq0, se0), (sp1, sq1, se1))
        ssca = (sa0, sa1)
        cid = lax.axis_index("c")
        sid = lax.axis_index("s")
        wid = sid * 2 + cid

        nbase = NCH // 32
        extra = NCH % 32
        cnt = nbase + jnp.where(wid < extra, 1, 0)
        cstart = wid * nbase + jnp.minimum(wid, extra)

        pltpu.sync_copy(dst_hbm.at[pl.ds(cstart, IH)], dst_all)
        pltpu.sync_copy(src_hbm.at[pl.ds(cstart, IH)], src_all)

        base_row = sid * RPT
        pltpu.sync_copy(zA_hbm.at[pl.ds(base_row, RPT)],
                        A_sh.at[pl.ds(base_row, RPT)])
        plsc.subcore_barrier()

        IR = KPT - IH

        def idxrow(k):
            return jnp.where(k < IH, k, k - IH)

        def issue(k, b):
            sp, sq, se = sems[b]
            r = idxrow(k)
            pltpu.async_copy(P_hbm.at[dst_all.at[r]], Pd.at[b], sp)
            pltpu.async_copy(Q_hbm.at[src_all.at[r]], Qs.at[b], sq)
            pltpu.async_copy(Eh_hbm.at[pl.ds((cstart + k) * CB, CB)],
                             Ehb.at[b], se)

        def wait(k, b):
            sp, sq, se = sems[b]
            r = idxrow(k)
            pltpu.make_async_copy(P_hbm.at[dst_all.at[r]], Pd.at[b], sp).wait()
            pltpu.make_async_copy(Q_hbm.at[src_all.at[r]], Qs.at[b], sq).wait()
            pltpu.make_async_copy(Eh_hbm.at[pl.ds((cstart + k) * CB, CB)],
                                  Ehb.at[b], se).wait()

        issue(0, 0)

        def pair_body(i, carry):
            for b in range(2):
                k = 2 * i + b

                @pl.when(k < cnt)
                def _():
                    wait(k, b)

                    @pl.when(k + 1 == IH)
                    def _():
                        pltpu.make_async_copy(
                            dst_hbm.at[pl.ds(cstart + IH, IR)],
                            dst_all.at[pl.ds(0, IR)], si0).wait()
                        pltpu.make_async_copy(
                            src_hbm.at[pl.ds(cstart + IH, IR)],
                            src_all.at[pl.ds(0, IR)], si1).wait()

                    @pl.when(k + 1 < cnt)
                    def _():
                        issue(k + 1, 1 - b)

                    @pl.when(k >= 2)
                    def _():
                        pltpu.make_async_copy(
                            Tb.at[b], A_sh.at[dst_all.at[0]], ssca[b]).wait()

                    mask_hi = jnp.full((16,), -65536, jnp.int32)
                    sixteen = jnp.full((16,), 16, jnp.int32)

                    def cvt(v):
                        vi = plsc.bitcast(v, jnp.int32)
                        lo = plsc.bitcast(jnp.left_shift(vi, sixteen), F32)
                        hi = plsc.bitcast(jnp.bitwise_and(vi, mask_hi), F32)
                        return lo, hi

                    @plsc.parallel_loop(0, CB, unroll=4)
                    def rbody(r):
                        for g in range(Dh // 16):
                            sl = pl.ds(g * 16, 16)
                            plo, phi = cvt(Pd[b, r, sl])
                            qlo, qhi = cvt(Qs[b, r, sl])
                            elo, ehi = cvt(Ehb[b, r, sl])
                            Tb[b, r, pl.ds(g * 16, 16)] = jnp.maximum(
                                plo + qlo + elo, 0.0)
                            Tb[b, r, pl.ds(Dh + g * 16, 16)] = jnp.maximum(
                                phi + qhi + ehi, 0.0)

                    ir = idxrow(k)
                    pltpu.async_copy(Tb.at[b], A_sh.at[dst_all.at[ir]],
                                     ssca[b])

                    @pl.when(k == RELOAD_K)
                    def _():
                        pltpu.async_copy(dst_hbm.at[pl.ds(cstart + IH, IR)],
                                         dst_all.at[pl.ds(0, IR)], si0)
                        pltpu.async_copy(src_hbm.at[pl.ds(cstart + IH, IR)],
                                         src_all.at[pl.ds(0, IR)], si1)
            return carry
        lax.fori_loop(0, (cnt + 1) // 2, pair_body, 0)
        for b in range(2):
            pltpu.make_async_copy(
                Tb.at[b], A_sh.at[dst_all.at[0]], ssca[b]).wait()
        plsc.subcore_barrier()

        pltpu.sync_copy(A_sh.at[pl.ds(base_row, RPT)],
                        A_out.at[cid, pl.ds(base_row, RPT)])

    run = pl.kernel(body, mesh=mesh,
                    out_type=[jax.ShapeDtypeStruct((2, N, D), F32)],
                    scratch_types=scratch,
                    compiler_params=pltpu.CompilerParams(
                        use_tc_tiling_on_sc=False,
                        needs_layout_passes=False))
    return run(Ppk, Qpk, Ehpk, dst2d, src2d, zeroA)


def _gr_update(x, A_part, Deg_part, W2T, b2, WihT, WhhT, bsum,
               WgT, bgp, WfT, bfp, WdT2=None, WsT2=None):
    N, D = x.shape
    BN = 2000
    GP = WgT.shape[1]
    emit_pq = WdT2 is not None
    nsteps = N // BN

    def body(x_ref, a2_ref, dg_ref, w2t, b2r, wih, whh, bs, wgt, bg, wft, bf,
             *refs):
        if emit_pq:
            wd2, ws2, x1_ref, hg_ref, p_ref, q_ref, acc = refs
        else:
            x1_ref, hg_ref, acc = refs
        i = pl.program_id(0)
        A = a2_ref[0] + a2_ref[1]
        deg = dg_ref[0, :, 0:1] + dg_ref[1, :, 0:1]
        a = jnp.dot(A, w2t[...], preferred_element_type=F32) + deg * b2r[...]
        xb = x_ref[...]
        g = (jnp.dot(xb, wih[...], preferred_element_type=F32)
             + jnp.dot(a, whh[...], preferred_element_type=F32) + bs[...])
        i_g = g[:, 0:D]
        g_g = g[:, 2 * D:3 * D]
        o_g = g[:, 3 * D:4 * D]
        c = jax.nn.sigmoid(i_g) * jnp.tanh(g_g)
        xn = jax.nn.sigmoid(o_g) * jnp.tanh(c)
        x1_ref[...] = xn
        if emit_pq:
            p_ref[...] = _pack_halves(
                jnp.dot(xn, wd2[...], preferred_element_type=F32))
            q_ref[...] = _pack_halves(
                jnp.dot(xn, ws2[...], preferred_element_type=F32))
        gate = jax.nn.sigmoid(jnp.dot(xn, wgt[...], preferred_element_type=F32) + bg[...])
        hv = jnp.dot(xn, wft[...], preferred_element_type=F32) + bf[...]
        contrib = jnp.sum(gate * hv, axis=0, keepdims=True)

        @pl.when(i == 0)
        def _():
            acc[...] = jnp.zeros_like(acc)

        acc[0:1, 0:GP] += contrib
        hg_ref[...] = acc[0:1, 0:GP]

    in_specs = [
        pl.BlockSpec((BN, D), lambda i: (i, 0)),
        pl.BlockSpec((2, BN, D), lambda i: (0, i, 0)),
        pl.BlockSpec((2, BN, 16), lambda i: (0, i, 0)),
        pl.BlockSpec((D, D), lambda i: (0, 0)),
        pl.BlockSpec((1, D), lambda i: (0, 0)),
        pl.BlockSpec((D, 4 * D), lambda i: (0, 0)),
        pl.BlockSpec((D, 4 * D), lambda i: (0, 0)),
        pl.BlockSpec((1, 4 * D), lambda i: (0, 0)),
        pl.BlockSpec((D, GP), lambda i: (0, 0)),
        pl.BlockSpec((1, GP), lambda i: (0, 0)),
        pl.BlockSpec((D, GP), lambda i: (0, 0)),
        pl.BlockSpec((1, GP), lambda i: (0, 0)),
    ]
    args = [x, A_part, Deg_part, W2T, b2, WihT, WhhT, bsum, WgT, bgp, WfT, bfp]
    out_specs = [pl.BlockSpec((BN, D), lambda i: (i, 0)),
                 pl.BlockSpec((1, GP), lambda i: (0, 0))]
    out_shape = [jax.ShapeDtypeStruct((N, D), F32),
                 jax.ShapeDtypeStruct((1, GP), F32)]
    if emit_pq:
        in_specs += [pl.BlockSpec((D, D), lambda i: (0, 0)),
                     pl.BlockSpec((D, D), lambda i: (0, 0))]
        args += [WdT2, WsT2]
        out_specs += [pl.BlockSpec((BN, D // 2), lambda i: (i, 0)),
                      pl.BlockSpec((BN, D // 2), lambda i: (i, 0))]
        out_shape += [jax.ShapeDtypeStruct((N, D // 2), F32),
                      jax.ShapeDtypeStruct((N, D // 2), F32)]

    return pl.pallas_call(
        body,
        grid=(nsteps,),
        in_specs=in_specs,
        out_specs=out_specs,
        out_shape=out_shape,
        scratch_shapes=[pltpu.VMEM((8, 128), F32)],
    )(*args)


def _head_call(h_G, hG_nn, x2, seq_node, seq_edge,
               fanWT, fan_b, A1, A2, nn_b, B1, B2, fae_b, C1, C2, fs_b):
    N, D = x2.shape

    def _pick(v, idx):
        it = lax.broadcasted_iota(jnp.int32, v.shape, 1)
        return jnp.sum(jnp.where(it == idx, v, 0.0))

    def body(seqn, seqe, hg_ref, hgnn_ref, fanwt, fanb, a1, a2, nnb,
             b1m, b2m, faeb, c1m, c2m, fsb, x2_hbm, out_ref, x2u, sem):
        u = seqe[0]
        ntype = seqn[1]
        bond = seqe[2]
        cp = pltpu.make_async_copy(x2_hbm.at[pl.ds(u, 1)], x2u, sem)
        cp.start()
        hg = hg_ref[...]
        hgnn = hgnn_ref[...]
        ntl = jnp.dot(hg, fanwt[...], preferred_element_type=F32) + fanb[...]
        m1 = jnp.max(ntl)
        p_nt = jnp.exp(ntl - m1)
        p_nt = p_nt / jnp.sum(p_nt)
        lp = jnp.log(_pick(p_nt, ntype)) + jnp.log(_pick(p_nt, 113))
        nne = (jnp.dot(hgnn, a1[...], preferred_element_type=F32)
               + jnp.dot(p_nt, a2[...], preferred_element_type=F32) + nnb[...])
        ael = (jnp.dot(hg, b1m[...], preferred_element_type=F32)
               + jnp.dot(nne, b2m[...], preferred_element_type=F32) + faeb[...])
        m2 = jnp.max(ael)
        p_ae = jnp.exp(ael - m2)
        p_ae = p_ae / jnp.sum(p_ae)
        lp = lp + jnp.log(_pick(p_ae, 1)) + jnp.log(_pick(p_ae, 0))
        cp.wait()
        sl = (jnp.dot(x2u[...], c1m[...], preferred_element_type=F32)
              + jnp.dot(nne, c2m[...], preferred_element_type=F32) + fsb[...])
        m3 = jnp.max(sl)
        p_s = jnp.exp(sl - m3)
        p_s = p_s / jnp.sum(p_s)
        lp = lp + jnp.log(_pick(p_s, bond))
        out_ref[0, 0] = lp

    return pl.pallas_call(
        body,
        in_specs=[
            pl.BlockSpec(memory_space=pltpu.SMEM),
            pl.BlockSpec(memory_space=pltpu.SMEM),
            pl.BlockSpec(memory_space=pltpu.VMEM),
            pl.BlockSpec(memory_space=pltpu.VMEM),
            pl.BlockSpec(memory_space=pltpu.VMEM),
            pl.BlockSpec(memory_space=pltpu.VMEM),
            pl.BlockSpec(memory_space=pltpu.VMEM),
            pl.BlockSpec(memory_space=pltpu.VMEM),
            pl.BlockSpec(memory_space=pltpu.VMEM),
            pl.BlockSpec(memory_space=pltpu.VMEM),
            pl.BlockSpec(memory_space=pltpu.VMEM),
            pl.BlockSpec(memory_space=pltpu.VMEM),
            pl.BlockSpec(memory_space=pltpu.VMEM),
            pl.BlockSpec(memory_space=pltpu.VMEM),
            pl.BlockSpec(memory_space=pltpu.VMEM),
            pl.BlockSpec(memory_space=pl.ANY),
        ],
        out_specs=pl.BlockSpec(memory_space=pltpu.SMEM),
        out_shape=jax.ShapeDtypeStruct((1, 1), F32),
        scratch_shapes=[pltpu.VMEM((1, D), F32), pltpu.SemaphoreType.DMA],
    )(seq_node, seq_edge, h_G, hG_nn, fanWT, fan_b, A1, A2, nn_b,
      B1, B2, fae_b, C1, C2, fs_b, x2)


def kernel(x, edge_attr, edge_index, seq_node, seq_edge,
           R_fe_W1, R_fe_b1, R_fe_W2, R_fe_b2, R_Wih, R_Whh, R_bih, R_bhh,
           R_Wg, R_bg, R_Wf, R_bf,
           NN_fe_W1, NN_fe_b1, NN_fe_W2, NN_fe_b2, NN_Wih, NN_Whh, NN_bih,
           NN_bhh, NN_Wg, NN_bg, NN_Wf, NN_bf,
           NE_fe_W1, NE_fe_b1, NE_fe_W2, NE_fe_b2, NE_Wih, NE_Whh, NE_bih,
           NE_bhh, NE_Wg, NE_bg, NE_Wf, NE_bf,
           fan_W, fan_b, fae_W, fae_b, fs_W, fs_b,
           nn_init_W, nn_init_b, ne_init_W, ne_init_b):
    N, D = x.shape
    E, DE = edge_attr.shape
    G = fan_W.shape[1]
    GP = 64

    def prep(W1, b1, W2, Wih, Whh, bih, bhh, Wg, bg, Wf, bf):
        WdT = W1[:, :D].T
        WsT = W1[:, D:2 * D].T
        WeT = W1[:, 2 * D:].T
        b1r = b1[None, :]
        W2T = W2.T
        WihT = Wih.T
        WhhT = Whh.T
        bsum = (bih + bhh)[None, :]
        WgT = jnp.zeros((D, GP), F32).at[:, :G].set(Wg.T)
        bgp = jnp.zeros((1, GP), F32).at[0, :G].set(bg)
        WfT = jnp.zeros((D, GP), F32).at[:, :G].set(Wf.T)
        bfp = jnp.zeros((1, GP), F32).at[0, :G].set(bf)
        return WdT, WsT, WeT, b1r, W2T, WihT, WhhT, bsum, WgT, bgp, WfT, bfp

    (R_WdT, R_WsT, R_WeT, R_b1r, R_W2T, R_WihT, R_WhhT, R_bsum,
     R_WgT, R_bgp, R_WfT, R_bfp) = prep(R_fe_W1, R_fe_b1, R_fe_W2, R_Wih,
                                        R_Whh, R_bih, R_bhh, R_Wg, R_bg,
                                        R_Wf, R_bf)
    (N_WdT, N_WsT, N_WeT, N_b1r, N_W2T, N_WihT, N_WhhT, N_bsum,
     N_WgT, N_bgp, N_WfT, N_bfp) = prep(NN_fe_W1, NN_fe_b1, NN_fe_W2, NN_Wih,
                                        NN_Whh, NN_bih, NN_bhh, NN_Wg, NN_bg,
                                        NN_Wf, NN_bf)

    def chunked_idx(cb):
        nch = E // cb
        kpt = -(-nch // 32)
        cpad = 31 * (nch // 32) + min(31, nch % 32) + kpt
        cpad = -(-cpad // 8) * 8
        pad = cpad * cb - E
        d2 = jnp.concatenate(
            [edge_index[1], jnp.zeros((pad,), jnp.int32)]).reshape(cpad, cb)
        s2 = jnp.concatenate(
            [edge_index[0], jnp.zeros((pad,), jnp.int32)]).reshape(cpad, cb)
        return d2, s2, nch

    dst2d, src2d, NCH = chunked_idx(32)
    dst2d64, src2d64, NCH64 = chunked_idx(64)
    zeroA = jnp.zeros((N, D), F32)
    zeroD = jnp.zeros((N, 16), F32)

    P_R, Q_R = _pq_call(x, R_WdT, R_WsT)
    Eh_R = _eh_call(edge_attr, R_WeT, R_b1r)
    A_R, Deg = _sc_edge_pass(P_R, Q_R, Eh_R, dst2d, src2d, zeroA, zeroD,
                             NCH, with_deg=True)
    Eh_NN = _eh_call(edge_attr, N_WeT, N_b1r, packed=True)
    x1, h_G, P_NN, Q_NN = _gr_update(
        x, A_R, Deg, R_W2T, R_fe_b2[None, :], R_WihT, R_WhhT, R_bsum,
        R_WgT, R_bgp, R_WfT, R_bfp, N_WdT, N_WsT)

    (A_NN,) = _sc_edge_pass_packed(P_NN, Q_NN, Eh_NN, dst2d64, src2d64,
                                   zeroA, NCH64)
    x2, hG_nn = _gr_update(
        x1, A_NN, Deg, N_W2T, NN_fe_b2[None, :], N_WihT, N_WhhT, N_bsum,
        N_WgT, N_bgp, N_WfT, N_bfp)

    fanWT = jnp.zeros((GP, 114), F32).at[:G].set(fan_W.T)
    A1 = jnp.zeros((GP, D), F32).at[:G].set(nn_init_W[:, :G].T)
    A2 = nn_init_W[:, G:].T
    B1 = jnp.zeros((GP, 2), F32).at[:G].set(fae_W[:, :G].T)
    B2 = fae_W[:, G:].T
    C1 = fs_W[:, :D].T
    C2 = fs_W[:, D:].T
    lp = _head_call(h_G, hG_nn, x2, seq_node, seq_edge,
                    fanWT, fan_b[None, :], A1, A2, nn_init_b[None, :],
                    B1, B2, fae_b[None, :], C1, C2, fs_b[None, :])
    return lp[0, 0]
